# baseline (speedup 1.0000x reference)
import functools

import jax
import jax.numpy as jnp
from jax.experimental import pallas as pl
from jax.experimental.pallas import tpu as pltpu
from jax.experimental.pallas import tpu_sc as plsc

TOPK = 5
NEG = -1e30


def _stream_body(n_docs, h_ref, wq_ref, bq_ref, docs_ref,
                 vals_ref, idx_ref, w_ref, qn_s, sc_all):
    i = pl.program_id(0)
    nblk = pl.num_programs(0)
    bb, _, d = h_ref.shape
    nb = docs_ref.shape[0]
    k = TOPK

    @pl.when(i == 0)
    def _query():
        h = h_ref[...]
        q = jnp.mean(h, axis=1)
        ql = jax.lax.dot_general(q, wq_ref[...], (((1,), (1,)), ((), ())),
                                 preferred_element_type=jnp.float32)
        ql = ql + bq_ref[...][None, :]
        s2q = jnp.sum(ql * ql, axis=1, keepdims=True)
        qn_s[...] = ql * jax.lax.rsqrt(jnp.maximum(s2q, 1e-24))

    x = docs_ref[...]
    raw = jax.lax.dot_general(qn_s[...], x, (((1,), (1,)), ((), ())),
                              preferred_element_type=jnp.float32)
    ones = jnp.ones((1, d), jnp.float32)
    s2 = jax.lax.dot_general(ones, x * x, (((1,), (1,)), ((), ())),
                             preferred_element_type=jnp.float32)
    sc = raw * jax.lax.rsqrt(jnp.maximum(s2, 1e-24))
    col = jax.lax.broadcasted_iota(jnp.int32, (bb, nb), 1)
    sc = jnp.where(col + i * nb < n_docs, sc, NEG)
    sc_all[:, pl.ds(pl.multiple_of(i * nb, nb), nb)] = sc

    @pl.when(i == nblk - 1)
    def _topk():
        ncols = sc_all.shape[1]
        lane = jax.lax.broadcasted_iota(jnp.int32, (bb, 128), 1)
        sv = jnp.full((bb, 128), NEG, jnp.float32)
        si = jnp.zeros((bb, 128), jnp.int32)
        for j in range(k):
            s = sc_all[...]
            ciota = jax.lax.broadcasted_iota(jnp.int32, (bb, ncols), 1)
            m = jnp.max(s, axis=1, keepdims=True)
            gi = jnp.min(jnp.where(s == m, ciota, 2147483647), axis=1,
                         keepdims=True)
            sv = jnp.where(lane == j, m, sv)
            si = jnp.where(lane == j, gi, si)
            sc_all[...] = jnp.where(ciota == gi, NEG, s)
        vals_ref[...] = sv[:, :k]
        idx_ref[...] = si[:, :k]
        e = jnp.where(lane < k, jnp.exp(sv - sv[:, :1]), 0.0)
        w = e / jnp.sum(e, axis=1, keepdims=True)
        w_ref[...] = w[:, :k]


def _sc_gather_body(idx_hbm, docs_hbm, out_hbm, idx_v, rows_v, sem):
    ci = jax.lax.axis_index("c")
    si = jax.lax.axis_index("s")

    @pl.when((ci == 0) & (si == 0))
    def _():
        pltpu.sync_copy(idx_hbm, idx_v)
        pltpu.async_copy(docs_hbm.at[idx_v], rows_v, sem).wait()
        pltpu.sync_copy(rows_v, out_hbm)


def _fuse_body(h_ref, wg_ref, rows_ref, w_ref, bg_ref, o_ref):
    d = h_ref.shape[2]
    k = w_ref.shape[2]
    h = h_ref[0]
    x = rows_ref[0, :k]
    s2 = jnp.sum(x * x, axis=1, keepdims=True)
    rn = x * jax.lax.rsqrt(jnp.maximum(s2, 1e-24))
    ctxv = jax.lax.dot_general(w_ref[0], rn, (((1,), (0,)), ((), ())),
                               preferred_element_type=jnp.float32)
    lg = jax.lax.dot_general(h, wg_ref[:, :d], (((1,), (1,)), ((), ())),
                             preferred_element_type=jnp.float32)
    ct = jax.lax.dot_general(ctxv, wg_ref[:, d:], (((1,), (1,)), ((), ())),
                             preferred_element_type=jnp.float32)
    z = lg + ct + bg_ref[...][None, :]
    g = jax.nn.sigmoid(z)
    o_ref[0] = g * h + (1.0 - g) * ctxv


def kernel(hidden_states, doc_embeddings, Wq, bq, Wg, bg):
    b, s, d = hidden_states.shape
    n, _ = doc_embeddings.shape
    k = TOPK
    nb = 4096
    nblk = (n + nb - 1) // nb
    bs = 512
    assert s % bs == 0

    vals, idxs, wts = pl.pallas_call(
        functools.partial(_stream_body, n),
        grid=(nblk,),
        in_specs=[
            pl.BlockSpec((b, s, d), lambda i: (0, 0, 0)),
            pl.BlockSpec((d, d), lambda i: (0, 0)),
            pl.BlockSpec((d,), lambda i: (0,)),
            pl.BlockSpec((nb, d), lambda i: (i, 0)),
        ],
        out_specs=[
            pl.BlockSpec((b, k), lambda i: (0, 0)),
            pl.BlockSpec((b, k), lambda i: (0, 0)),
            pl.BlockSpec((b, k), lambda i: (0, 0)),
        ],
        out_shape=[
            jax.ShapeDtypeStruct((b, k), jnp.float32),
            jax.ShapeDtypeStruct((b, k), jnp.int32),
            jax.ShapeDtypeStruct((b, k), jnp.float32),
        ],
        scratch_shapes=[
            pltpu.VMEM((b, d), jnp.float32),
            pltpu.VMEM((b, nblk * nb), jnp.float32),
        ],
        compiler_params=pltpu.CompilerParams(
            vmem_limit_bytes=116 * 1024 * 1024),
    )(hidden_states, Wq, bq, doc_embeddings)

    idx16 = jnp.zeros((b, 16 // b), jnp.int32).at[:, :k].set(idxs).reshape(16)

    rows16 = pl.kernel(
        _sc_gather_body,
        mesh=plsc.VectorSubcoreMesh(core_axis_name="c", subcore_axis_name="s"),
        out_type=jax.ShapeDtypeStruct((16, d), jnp.float32),
        scratch_types=[
            pltpu.VMEM((16,), jnp.int32),
            pltpu.VMEM((16, d), jnp.float32),
            pltpu.SemaphoreType.DMA,
        ],
    )(idx16, doc_embeddings)

    fused = pl.pallas_call(
        _fuse_body,
        grid=(b, s // bs),
        in_specs=[
            pl.BlockSpec((1, bs, d), lambda bi, si: (bi, si, 0)),
            pl.BlockSpec((d, 2 * d), lambda bi, si: (0, 0)),
            pl.BlockSpec((1, 16 // b, d), lambda bi, si: (bi, 0, 0)),
            pl.BlockSpec((1, 1, k), lambda bi, si: (bi, 0, 0)),
            pl.BlockSpec((d,), lambda bi, si: (0,)),
        ],
        out_specs=pl.BlockSpec((1, bs, d), lambda bi, si: (bi, si, 0)),
        out_shape=jax.ShapeDtypeStruct((b, s, d), jnp.float32),
    )(hidden_states, Wg, rows16.reshape(b, 16 // b, d),
      wts.reshape(b, 1, k), bg)

    return vals, idxs, fused


# speedup vs baseline: 1.0075x; 1.0075x over previous
import functools

import jax
import jax.numpy as jnp
from jax.experimental import pallas as pl
from jax.experimental.pallas import tpu as pltpu
from jax.experimental.pallas import tpu_sc as plsc

TOPK = 5
NEG = -1e30


def _stream_body(n_docs, h_ref, wq_ref, bq_ref, docs_ref,
                 vals_ref, idx_ref, w_ref, idxp_ref, qn_s, sc_all):
    i = pl.program_id(0)
    nblk = pl.num_programs(0)
    bb, _, d = h_ref.shape
    nb = docs_ref.shape[0]
    k = TOPK

    @pl.when(i == 0)
    def _query():
        h = h_ref[...]
        q = jnp.mean(h, axis=1)
        ql = jax.lax.dot_general(q, wq_ref[...], (((1,), (1,)), ((), ())),
                                 preferred_element_type=jnp.float32)
        ql = ql + bq_ref[...][None, :]
        s2q = jnp.sum(ql * ql, axis=1, keepdims=True)
        qn_s[...] = ql * jax.lax.rsqrt(jnp.maximum(s2q, 1e-24))

    x = docs_ref[...]
    raw = jax.lax.dot_general(qn_s[...], x, (((1,), (1,)), ((), ())),
                              preferred_element_type=jnp.float32)
    ones = jnp.ones((1, d), jnp.float32)
    s2 = jax.lax.dot_general(ones, x * x, (((1,), (1,)), ((), ())),
                             preferred_element_type=jnp.float32)
    sc = raw * jax.lax.rsqrt(jnp.maximum(s2, 1e-24))
    col = jax.lax.broadcasted_iota(jnp.int32, (bb, nb), 1)
    sc = jnp.where(col + i * nb < n_docs, sc, NEG)
    sc_all[:, pl.ds(pl.multiple_of(i * nb, nb), nb)] = sc

    @pl.when(i == nblk - 1)
    def _topk():
        ncols = sc_all.shape[1]
        lane = jax.lax.broadcasted_iota(jnp.int32, (bb, 128), 1)
        sv = jnp.full((bb, 128), NEG, jnp.float32)
        si = jnp.zeros((bb, 128), jnp.int32)
        for j in range(k):
            s = sc_all[...]
            ciota = jax.lax.broadcasted_iota(jnp.int32, (bb, ncols), 1)
            m = jnp.max(s, axis=1, keepdims=True)
            gi = jnp.min(jnp.where(s == m, ciota, 2147483647), axis=1,
                         keepdims=True)
            sv = jnp.where(lane == j, m, sv)
            si = jnp.where(lane == j, gi, si)
            sc_all[...] = jnp.where(ciota == gi, NEG, s)
        vals_ref[...] = sv[:, :k]
        idx_ref[...] = si[:, :k]
        idxp_ref[...] = si[:, :idxp_ref.shape[1]]
        e = jnp.where(lane < k, jnp.exp(sv - sv[:, :1]), 0.0)
        w = e / jnp.sum(e, axis=1, keepdims=True)
        w_ref[...] = w[:, :k]


def _sc_gather_body(idx_hbm, docs_hbm, out_hbm, idx_v, rows_v, sem):
    ci = jax.lax.axis_index("c")
    si = jax.lax.axis_index("s")

    @pl.when((ci == 0) & (si == 0))
    def _():
        pltpu.sync_copy(idx_hbm, idx_v)
        pltpu.async_copy(docs_hbm.at[idx_v], rows_v, sem).wait()
        pltpu.sync_copy(rows_v, out_hbm)


def _fuse_body(h_ref, wg_ref, rows_ref, w_ref, bg_ref, o_ref):
    d = h_ref.shape[2]
    k = w_ref.shape[2]
    h = h_ref[0]
    x = rows_ref[0, :k]
    s2 = jnp.sum(x * x, axis=1, keepdims=True)
    rn = x * jax.lax.rsqrt(jnp.maximum(s2, 1e-24))
    ctxv = jax.lax.dot_general(w_ref[0], rn, (((1,), (0,)), ((), ())),
                               preferred_element_type=jnp.float32)
    lg = jax.lax.dot_general(h, wg_ref[:, :d], (((1,), (1,)), ((), ())),
                             preferred_element_type=jnp.float32)
    ct = jax.lax.dot_general(ctxv, wg_ref[:, d:], (((1,), (1,)), ((), ())),
                             preferred_element_type=jnp.float32)
    z = lg + ct + bg_ref[...][None, :]
    g = jax.nn.sigmoid(z)
    o_ref[0] = g * h + (1.0 - g) * ctxv


def kernel(hidden_states, doc_embeddings, Wq, bq, Wg, bg):
    b, s, d = hidden_states.shape
    n, _ = doc_embeddings.shape
    k = TOPK
    nb = 4096
    nblk = (n + nb - 1) // nb
    bs = 512
    assert s % bs == 0

    vals, idxs, wts, idxp = pl.pallas_call(
        functools.partial(_stream_body, n),
        grid=(nblk,),
        in_specs=[
            pl.BlockSpec((b, s, d), lambda i: (0, 0, 0)),
            pl.BlockSpec((d, d), lambda i: (0, 0)),
            pl.BlockSpec((d,), lambda i: (0,)),
            pl.BlockSpec((nb, d), lambda i: (i, 0)),
        ],
        out_specs=[
            pl.BlockSpec((b, k), lambda i: (0, 0)),
            pl.BlockSpec((b, k), lambda i: (0, 0)),
            pl.BlockSpec((b, k), lambda i: (0, 0)),
            pl.BlockSpec((b, 16 // b), lambda i: (0, 0)),
        ],
        out_shape=[
            jax.ShapeDtypeStruct((b, k), jnp.float32),
            jax.ShapeDtypeStruct((b, k), jnp.int32),
            jax.ShapeDtypeStruct((b, k), jnp.float32),
            jax.ShapeDtypeStruct((b, 16 // b), jnp.int32),
        ],
        scratch_shapes=[
            pltpu.VMEM((b, d), jnp.float32),
            pltpu.VMEM((b, nblk * nb), jnp.float32),
        ],
        compiler_params=pltpu.CompilerParams(
            vmem_limit_bytes=116 * 1024 * 1024),
    )(hidden_states, Wq, bq, doc_embeddings)

    idx16 = idxp.reshape(16)

    rows16 = pl.kernel(
        _sc_gather_body,
        mesh=plsc.VectorSubcoreMesh(core_axis_name="c", subcore_axis_name="s"),
        out_type=jax.ShapeDtypeStruct((16, d), jnp.float32),
        scratch_types=[
            pltpu.VMEM((16,), jnp.int32),
            pltpu.VMEM((16, d), jnp.float32),
            pltpu.SemaphoreType.DMA,
        ],
    )(idx16, doc_embeddings)

    fused = pl.pallas_call(
        _fuse_body,
        grid=(b, s // bs),
        in_specs=[
            pl.BlockSpec((1, bs, d), lambda bi, si: (bi, si, 0)),
            pl.BlockSpec((d, 2 * d), lambda bi, si: (0, 0)),
            pl.BlockSpec((1, 16 // b, d), lambda bi, si: (bi, 0, 0)),
            pl.BlockSpec((1, 1, k), lambda bi, si: (bi, 0, 0)),
            pl.BlockSpec((d,), lambda bi, si: (0,)),
        ],
        out_specs=pl.BlockSpec((1, bs, d), lambda bi, si: (bi, si, 0)),
        out_shape=jax.ShapeDtypeStruct((b, s, d), jnp.float32),
    )(hidden_states, Wg, rows16.reshape(b, 16 // b, d),
      wts.reshape(b, 1, k), bg)

    return vals, idxs, fused
